# baseline (speedup 1.0000x reference)
import functools
import math

import jax
import jax.numpy as jnp
from jax import lax
from jax.experimental import pallas as pl
from jax.experimental.pallas import tpu as pltpu
from jax.experimental.pallas import tpu_sc as plsc

_B, _L, _N = 4, 4, 1024
_DIM, _HEADS, _DH = 256, 8, 32
_INNER = _HEADS * _DH
_NS = 8
_R2 = 0.2 * 0.2
_SCALE = _DH ** -0.5

_ROWS = _B * _L * _N * _NS
_TBL = _B * _L * _N

_NC, _NSC = 2, 16
_NW = _NC * _NSC
_RPW = _ROWS // _NW
_CH = 64


def _ln_qkv_body(f_ref, g_ref, b_ref, wt_ref, kv_ref, q_ref):
    x = f_ref[0]
    mu = jnp.mean(x, axis=-1, keepdims=True)
    var = jnp.mean((x - mu) ** 2, axis=-1, keepdims=True)
    nf = (x - mu) / jnp.sqrt(var + 1e-5) * g_ref[0] + b_ref[0]
    qkv = jnp.dot(nf.astype(jnp.bfloat16), wt_ref[...],
                  preferred_element_type=jnp.float32).astype(jnp.bfloat16)
    q_ref[0] = qkv[:, 0:_INNER]
    kv_ref[0] = qkv[:, _INNER:3 * _INNER]


def _index_body(xyz_ref, xyzt_ref, idx_ref):
    bi = pl.program_id(0)
    xyz = xyz_ref[0]
    xq = xyz[_L - 1]
    sqq = jnp.sum(xq * xq, axis=-1, keepdims=True)

    ii = jax.lax.broadcasted_iota(jnp.int32, (_N, _N), 0)
    jj = jax.lax.broadcasted_iota(jnp.int32, (_N, _N), 1)
    tri16 = jnp.where(ii <= jj, 1.0, 0.0).astype(jnp.bfloat16)
    jrow = jax.lax.broadcasted_iota(jnp.int32, (_N, 128), 0)
    crow = jax.lax.broadcasted_iota(jnp.int32, (_N, 128), 1)
    jmat16 = jnp.where(crow == 0, (jrow // 4).astype(jnp.float32),
                       jnp.where(crow == 1, (jrow % 4).astype(jnp.float32),
                                 0.0)).astype(jnp.bfloat16)

    rr = jax.lax.broadcasted_iota(jnp.int32, (_N, _NS), 1).astype(jnp.float32)

    for l in range(_L):
        xs = xyz[l]
        sqs = jnp.sum(xs * xs, axis=-1)[None, :]
        d2 = (sqq + sqs) - 2.0 * jnp.dot(xq, xyzt_ref[0, l],
                                         preferred_element_type=jnp.float32)
        mask16 = jnp.where(d2 < _R2, 1.0, 0.0).astype(jnp.bfloat16)
        grank = jnp.dot(mask16, tri16, preferred_element_type=jnp.float32)
        gm = (grank * mask16.astype(jnp.float32)).astype(jnp.bfloat16)
        cnt = grank[:, _N - 1:_N]

        cols = []
        for r in range(1, _NS + 1):
            oh16 = jnp.where(gm == jnp.full((), r, jnp.bfloat16),
                             jnp.full((), 1, jnp.bfloat16),
                             jnp.full((), 0, jnp.bfloat16))
            c = jnp.dot(oh16, jmat16, preferred_element_type=jnp.float32)
            cols.append(4.0 * c[:, 0:1] + c[:, 1:2])
        idxm = jnp.concatenate(cols, axis=-1)
        first = idxm[:, 0:1]
        has = cnt > 0.0
        found = cnt > rr
        final = jnp.where(found, idxm,
                          jnp.where(has, first, jnp.zeros((), jnp.float32)))
        idx_ref[0, l] = final.astype(jnp.int32) + (bi * _L + l) * _N


def _sc_gather(kv_hbm, xyz_hbm, idx_hbm, gkv_hbm, gxyz_hbm,
               idx_v, kv_v, xyz_v, sem1, sem2):
    wid = lax.axis_index("s") * _NC + lax.axis_index("c")
    base = wid * _RPW

    def chunk(i, carry):
        off = base + i * _CH
        pltpu.sync_copy(idx_hbm.at[pl.ds(off, _CH)], idx_v)
        c1 = pltpu.async_copy(kv_hbm.at[idx_v], kv_v, sem1)
        c2 = pltpu.async_copy(xyz_hbm.at[idx_v], xyz_v, sem2)
        c1.wait()
        c2.wait()
        pltpu.sync_copy(kv_v, gkv_hbm.at[pl.ds(off, _CH)])
        pltpu.sync_copy(xyz_v, gxyz_hbm.at[pl.ds(off, _CH)])
        return carry

    lax.fori_loop(0, _RPW // _CH, chunk, 0)


def _attn_body(gkv_ref, gxyz_ref, q3_ref, xq_ref, wsp_ref, wot_ref, bo_ref,
               o_ref, m_ref, z_ref, av_ref, da_ref):
    li = pl.program_id(1)
    q3 = q3_ref[0, 0].astype(jnp.float32)
    xq = xq_ref[0, 0]
    seg = (jax.lax.broadcasted_iota(jnp.int32, (_DIM, _HEADS), 0) // _DH ==
           jax.lax.broadcasted_iota(jnp.int32, (_DIM, _HEADS), 1)
           ).astype(jnp.float32)
    seg_t = seg.T

    @pl.when(li == 0)
    def _():
        m_ref[...] = jnp.full((_N, _HEADS), -1e30, jnp.float32)
        z_ref[...] = jnp.zeros((_N, _HEADS), jnp.float32)
        av_ref[...] = jnp.zeros((_N, _INNER), jnp.float32)
        da_ref[...] = jnp.full((3, _N, _HEADS), -1e30, jnp.float32)

    gkv = gkv_ref[0, 0]
    gxyz = gxyz_ref[0, 0]
    for s in range(_NS):
        kg = gkv[:, s * 512:s * 512 + _INNER].astype(jnp.float32)
        vg = gkv[:, s * 512 + _INNER:s * 512 + 2 * _INNER].astype(jnp.float32)
        xg = gxyz[:, s * 128:s * 128 + 3]
        sc = jnp.dot(kg * q3, seg,
                     preferred_element_type=jnp.float32) * _SCALE
        m_old = m_ref[...]
        m_new = jnp.maximum(m_old, sc)
        e = jnp.exp(sc - m_new)
        r = jnp.exp(m_old - m_new)
        m_ref[...] = m_new
        z_ref[...] = z_ref[...] * r + e
        r_exp = jnp.dot(r, seg_t, preferred_element_type=jnp.float32)
        e_exp = jnp.dot(e, seg_t, preferred_element_type=jnp.float32)
        av_ref[...] = av_ref[...] * r_exp + e_exp * vg
        for d in range(3):
            cur = e * (xg[:, d:d + 1] - xq[:, d:d + 1])
            if s == 0:
                da_ref[d] = jnp.where(li == 0, cur,
                                      jnp.maximum(da_ref[d] * r, cur))
            else:
                da_ref[d] = jnp.maximum(da_ref[d] * r, cur)

    @pl.when(li == _L - 1)
    def _():
        inv_z = 1.0 / z_ref[...]
        av = av_ref[...] * jnp.dot(inv_z, seg_t,
                                   preferred_element_type=jnp.float32)
        dis = jnp.zeros((_N, _INNER), jnp.float32)
        for d in range(3):
            dad = da_ref[d] * inv_z
            dis = dis + jnp.dot(dad, seg_t,
                                preferred_element_type=jnp.float32) * \
                wsp_ref[d:d + 1, :]
        y = jnp.dot(av + dis, wot_ref[...],
                    preferred_element_type=jnp.float32) + bo_ref[0]
        o_ref[0] = y * 0.5 * (1.0 + jax.lax.erf(y * (1.0 / math.sqrt(2.0))))


def _resid_body(g_ref, f_ref, o_ref):
    o_ref[0] = g_ref[0] + f_ref[0]


@jax.jit
def kernel(xyzs, feature, ln_g, ln_b, W_qkv, W_sp, W_out, b_out):
    b, l, n, dim = feature.shape
    ff = feature.reshape(b * l, n, dim)
    kv, q = pl.pallas_call(
        _ln_qkv_body,
        grid=(b * l,),
        in_specs=[
            pl.BlockSpec((1, n, dim), lambda i: (i, 0, 0)),
            pl.BlockSpec((1, dim), lambda i: (0, 0)),
            pl.BlockSpec((1, dim), lambda i: (0, 0)),
            pl.BlockSpec((dim, 3 * _INNER), lambda i: (0, 0)),
        ],
        out_specs=[
            pl.BlockSpec((1, n, 2 * _INNER), lambda i: (i, 0, 0)),
            pl.BlockSpec((1, n, _INNER), lambda i: (i, 0, 0)),
        ],
        out_shape=[
            jax.ShapeDtypeStruct((b * l, n, 2 * _INNER), jnp.bfloat16),
            jax.ShapeDtypeStruct((b * l, n, _INNER), jnp.bfloat16),
        ],
    )(ff, ln_g.reshape(1, dim), ln_b.reshape(1, dim),
      W_qkv.T.astype(jnp.bfloat16))

    xyzs_f = xyzs.reshape(b, l, n, 3)
    xyzs_t = jnp.swapaxes(xyzs_f, 2, 3)

    idx = pl.pallas_call(
        _index_body,
        grid=(b,),
        in_specs=[
            pl.BlockSpec((1, l, n, 3), lambda i: (i, 0, 0, 0)),
            pl.BlockSpec((1, l, 3, n), lambda i: (i, 0, 0, 0)),
        ],
        out_specs=pl.BlockSpec((1, l, n, _NS), lambda i: (i, 0, 0, 0)),
        out_shape=jax.ShapeDtypeStruct((b, l, n, _NS), jnp.int32),
    )(xyzs_f, xyzs_t)

    kv_i32 = jax.lax.bitcast_convert_type(
        kv.reshape(_TBL, _INNER, 2), jnp.int32)
    xyz_pad = jnp.pad(xyzs_f.reshape(_TBL, 3), ((0, 0), (0, 125)))
    idx_flat = idx.reshape(_ROWS)

    gkv_i32, gxyz = pl.kernel(
        _sc_gather,
        mesh=plsc.VectorSubcoreMesh(core_axis_name="c", subcore_axis_name="s"),
        out_type=[
            jax.ShapeDtypeStruct((_ROWS, _INNER), jnp.int32),
            jax.ShapeDtypeStruct((_ROWS, 128), jnp.float32),
        ],
        scratch_types=[
            pltpu.VMEM((_CH,), jnp.int32),
            pltpu.VMEM((_CH, _INNER), jnp.int32),
            pltpu.VMEM((_CH, 128), jnp.float32),
            pltpu.SemaphoreType.DMA,
            pltpu.SemaphoreType.DMA,
        ],
    )(kv_i32, xyz_pad, idx_flat)
    gkv = jax.lax.bitcast_convert_type(gkv_i32, jnp.bfloat16)

    wsp_tiled = jnp.tile(W_sp.T, (1, _HEADS))
    g_out = pl.pallas_call(
        _attn_body,
        grid=(b, l),
        in_specs=[
            pl.BlockSpec((1, 1, n, _NS * 512), lambda i, j: (i, j, 0, 0)),
            pl.BlockSpec((1, 1, n, _NS * 128), lambda i, j: (i, j, 0, 0)),
            pl.BlockSpec((1, 1, n, _INNER), lambda i, j: (i, l - 1, 0, 0)),
            pl.BlockSpec((1, 1, n, 3), lambda i, j: (i, l - 1, 0, 0)),
            pl.BlockSpec((3, _INNER), lambda i, j: (0, 0)),
            pl.BlockSpec((dim, dim), lambda i, j: (0, 0)),
            pl.BlockSpec((1, dim), lambda i, j: (0, 0)),
        ],
        out_specs=pl.BlockSpec((1, n, dim), lambda i, j: (i, 0, 0)),
        out_shape=jax.ShapeDtypeStruct((b, n, dim), jnp.float32),
        scratch_shapes=[
            pltpu.VMEM((n, _HEADS), jnp.float32),
            pltpu.VMEM((n, _HEADS), jnp.float32),
            pltpu.VMEM((n, _INNER), jnp.float32),
            pltpu.VMEM((3, n, _HEADS), jnp.float32),
        ],
    )(gkv.reshape(b, l, n, _NS * 512), gxyz.reshape(b, l, n, _NS * 128),
      q.reshape(b, l, n, _INNER), xyzs_f, wsp_tiled, W_out.T,
      b_out.reshape(1, dim))

    out = pl.pallas_call(
        _resid_body,
        grid=(b * l,),
        in_specs=[
            pl.BlockSpec((1, n, dim), lambda i: (i // l, 0, 0)),
            pl.BlockSpec((1, n, dim), lambda i: (i, 0, 0)),
        ],
        out_specs=pl.BlockSpec((1, n, dim), lambda i: (i, 0, 0)),
        out_shape=jax.ShapeDtypeStruct((b * l, n, dim), jnp.float32),
    )(g_out, ff)
    return out.reshape(b, l, n, dim)
